# Initial kernel scaffold; baseline (speedup 1.0000x reference)
import jax
import jax.numpy as jnp
from jax.experimental import pallas as pl


def kernel(feat, in_norm, out_norm, edge_index, W, b):
    raise NotImplementedError("write your pallas kernel here")


# speedup vs baseline: 3.4978x; 3.4978x over previous
import functools

import jax
import jax.numpy as jnp
from jax import lax
from jax.experimental import pallas as pl
from jax.experimental.pallas import tpu as pltpu
from jax.experimental.pallas import tpu_sc as plsc

N = 10000
D = 128
E = 320000

NC = 2
NS = 16
NW = NC * NS
CHUNK = 128
CPW = 80
E_PAD = NW * CPW * CHUNK
TRASH = N
SP_ROWS = 10112
PER_TILE = SP_ROWS // NS


def _matmul_body(feat_ref, onorm_ref, w_ref, p_ref):
    x = feat_ref[...] / onorm_ref[...]
    p_ref[...] = lax.dot_general(
        x, w_ref[...], (((1,), (1,)), ((), ())),
        preferred_element_type=jnp.float32)


def _scaled_matmul(feat, out_norm, W):
    grid = 10
    rb = N // grid
    return pl.pallas_call(
        _matmul_body,
        grid=(grid,),
        in_specs=[
            pl.BlockSpec((rb, D), lambda i: (i, 0)),
            pl.BlockSpec((rb, 1), lambda i: (i, 0)),
            pl.BlockSpec((D, D), lambda i: (0, 0)),
        ],
        out_specs=pl.BlockSpec((rb, D), lambda i: (i, 0)),
        out_shape=jax.ShapeDtypeStruct((N, D), jnp.float32),
    )(feat, out_norm.reshape(N, 1), W)


def _finish_body(parts_ref, inorm_ref, b_ref, out_ref):
    s = parts_ref[0] + parts_ref[1]
    out_ref[...] = s / inorm_ref[...] + b_ref[...]


def _finish(parts, in_norm, b):
    grid = 10
    rb = N // grid
    return pl.pallas_call(
        _finish_body,
        grid=(grid,),
        in_specs=[
            pl.BlockSpec((2, rb, D), lambda i: (0, i, 0)),
            pl.BlockSpec((rb, 1), lambda i: (i, 0)),
            pl.BlockSpec((1, D), lambda i: (0, 0)),
        ],
        out_specs=pl.BlockSpec((rb, D), lambda i: (i, 0)),
        out_shape=jax.ShapeDtypeStruct((N, D), jnp.float32),
    )(parts, in_norm.reshape(N, 1), b.reshape(1, D))


def _sc_body(p_hbm, src_hbm, dst_hbm, zeros_hbm, out_hbm,
             src_v, dst_v, buf, spmem, sem):
    cid = lax.axis_index("c")
    sid = lax.axis_index("s")
    wid = sid * NC + cid

    pltpu.sync_copy(zeros_hbm.at[pl.ds(sid * PER_TILE, PER_TILE)],
                    spmem.at[pl.ds(sid * PER_TILE, PER_TILE)])

    base = wid * CPW
    pltpu.sync_copy(src_hbm.at[pl.ds(base, CPW)], src_v)
    pltpu.sync_copy(dst_hbm.at[pl.ds(base, CPW)], dst_v)

    plsc.subcore_barrier()

    def step(j, carry):
        pltpu.async_copy(p_hbm.at[src_v.at[j]], buf, sem).wait()
        pltpu.sync_copy(buf, spmem.at[dst_v.at[j]], add=True)
        return carry

    lax.fori_loop(0, CPW, step, 0)

    plsc.subcore_barrier()

    pltpu.sync_copy(spmem.at[pl.ds(sid * PER_TILE, PER_TILE)],
                    out_hbm.at[cid].at[pl.ds(sid * PER_TILE, PER_TILE)])


def _sc_aggregate(p, src2d, dst2d, zeros):
    mesh = plsc.VectorSubcoreMesh(core_axis_name="c", subcore_axis_name="s",
                                  num_cores=NC, num_subcores=NS)
    k = pl.kernel(
        _sc_body,
        out_type=jax.ShapeDtypeStruct((NC, SP_ROWS, D), jnp.float32),
        mesh=mesh,
        scratch_types=[
            pltpu.VMEM((CPW, CHUNK), jnp.int32),
            pltpu.VMEM((CPW, CHUNK), jnp.int32),
            pltpu.VMEM((CHUNK, D), jnp.float32),
            pltpu.VMEM_SHARED((SP_ROWS, D), jnp.float32),
            pltpu.SemaphoreType.DMA,
        ],
    )
    return k(p, src2d, dst2d, zeros)


@jax.jit
def kernel(feat, in_norm, out_norm, edge_index, W, b):
    p = _scaled_matmul(feat, out_norm, W)
    pad = E_PAD - E
    src = jnp.concatenate([edge_index[0], jnp.zeros((pad,), jnp.int32)])
    dst = jnp.concatenate([edge_index[1], jnp.full((pad,), TRASH, jnp.int32)])
    src2d = src.reshape(E_PAD // CHUNK, CHUNK)
    dst2d = dst.reshape(E_PAD // CHUNK, CHUNK)
    zeros = jnp.zeros((SP_ROWS, D), jnp.float32)
    parts = _sc_aggregate(p, src2d, dst2d, zeros)
    return _finish(parts, in_norm, b)
